# baseline (speedup 1.0000x reference)
import functools

import jax
import jax.numpy as jnp
from jax import lax
from jax.experimental import pallas as pl
from jax.experimental.pallas import tpu as pltpu
from jax.experimental.pallas import tpu_sc as plsc

_EPS = 1e-5


def _seg_sum_sc(x_pad, src, dst, zeros, n_pad, num_edges):
    num_workers = 32
    tiles = 16
    chunk = 6400
    total_chunks = num_edges // chunk
    q, r = divmod(total_chunks, num_workers)
    all_ch = q + (1 if r else 0)
    all_ch += all_ch % 2
    slice_n = n_pad // tiles

    mesh = plsc.VectorSubcoreMesh(core_axis_name="c", subcore_axis_name="s")

    @functools.partial(
        pl.kernel,
        mesh=mesh,
        out_type=[
            jax.ShapeDtypeStruct((n_pad,), jnp.float32),
            jax.ShapeDtypeStruct((n_pad,), jnp.float32),
        ],
        scratch_types=[
            pltpu.VMEM((chunk,), jnp.int32),
            pltpu.VMEM((chunk,), jnp.int32),
            pltpu.VMEM((chunk,), jnp.int32),
            pltpu.VMEM((chunk,), jnp.int32),
            pltpu.VMEM((chunk,), jnp.float32),
            pltpu.VMEM((chunk,), jnp.float32),
            pltpu.VMEM_SHARED((n_pad,), jnp.float32),
            pltpu.SemaphoreType.DMA,
            pltpu.SemaphoreType.DMA,
            pltpu.SemaphoreType.DMA,
            pltpu.SemaphoreType.DMA,
            pltpu.SemaphoreType.DMA,
            pltpu.SemaphoreType.DMA,
        ],
    )
    def seg_kernel(x_hbm, src_hbm, dst_hbm, z_hbm, p0_hbm, p1_hbm,
                   srcbuf0, srcbuf1, dstbuf0, dstbuf1, valbuf0, valbuf1,
                   acc_sh, sem_i0, sem_i1, sem_g0, sem_g1, sem_s0, sem_s1):
        sc = lax.axis_index("c")
        sid = lax.axis_index("s")
        base_n = sid * slice_n
        pltpu.sync_copy(z_hbm.at[pl.ds(base_n, slice_n)],
                        acc_sh.at[pl.ds(base_n, slice_n)])
        plsc.subcore_barrier()

        wid = sc * tiles + sid
        nch = q + jnp.where(wid < r, 1, 0)

        def chunk_off(j):
            joff = jnp.where(j < nch, j, 0)
            return (wid + joff * num_workers) * chunk

        pltpu.sync_copy(z_hbm.at[pl.ds(0, chunk)], valbuf0)
        pltpu.sync_copy(z_hbm.at[pl.ds(0, chunk)], valbuf1)
        pltpu.sync_copy(dst_hbm.at[pl.ds(wid * chunk, chunk)], dstbuf0)
        pltpu.sync_copy(dst_hbm.at[pl.ds(wid * chunk, chunk)], dstbuf1)
        pltpu.async_copy(valbuf0, acc_sh.at[dstbuf0], sem_s0, add=True)
        pltpu.async_copy(valbuf1, acc_sh.at[dstbuf1], sem_s1, add=True)

        def body(k, _):
            j0 = 2 * k
            j1 = 2 * k + 1
            pltpu.make_async_copy(valbuf0, acc_sh.at[dstbuf0], sem_s0).wait()
            off0 = chunk_off(j0)
            ds0 = pltpu.async_copy(src_hbm.at[pl.ds(off0, chunk)], srcbuf0,
                                   sem_i0)
            dd0 = pltpu.async_copy(dst_hbm.at[pl.ds(off0, chunk)], dstbuf0,
                                   sem_i0)
            pltpu.make_async_copy(valbuf1, acc_sh.at[dstbuf1], sem_s1).wait()
            off1 = chunk_off(j1)
            ds1 = pltpu.async_copy(src_hbm.at[pl.ds(off1, chunk)], srcbuf1,
                                   sem_i1)
            dd1 = pltpu.async_copy(dst_hbm.at[pl.ds(off1, chunk)], dstbuf1,
                                   sem_i1)
            ds0.wait()
            dd0.wait()
            g0 = pltpu.async_copy(x_hbm.at[srcbuf0], valbuf0, sem_g0)
            ds1.wait()
            dd1.wait()
            g1 = pltpu.async_copy(x_hbm.at[srcbuf1], valbuf1, sem_g1)
            g0.wait()

            @pl.when(j0 >= nch)
            def _():
                pltpu.sync_copy(z_hbm.at[pl.ds(0, chunk)], valbuf0)

            pltpu.async_copy(valbuf0, acc_sh.at[dstbuf0], sem_s0, add=True)
            g1.wait()

            @pl.when(j1 >= nch)
            def _():
                pltpu.sync_copy(z_hbm.at[pl.ds(0, chunk)], valbuf1)

            pltpu.async_copy(valbuf1, acc_sh.at[dstbuf1], sem_s1, add=True)
            return _

        lax.fori_loop(0, all_ch // 2, body, None)
        pltpu.make_async_copy(valbuf0, acc_sh.at[dstbuf0], sem_s0).wait()
        pltpu.make_async_copy(valbuf1, acc_sh.at[dstbuf1], sem_s1).wait()
        plsc.subcore_barrier()

        @pl.when(sc == 0)
        def _():
            pltpu.sync_copy(acc_sh.at[pl.ds(base_n, slice_n)],
                            p0_hbm.at[pl.ds(base_n, slice_n)])

        @pl.when(sc == 1)
        def _():
            pltpu.sync_copy(acc_sh.at[pl.ds(base_n, slice_n)],
                            p1_hbm.at[pl.ds(base_n, slice_n)])

    return seg_kernel(x_pad, src, dst, zeros)


def _h0_pass(p0, p1, xw, wrel, brel, wroot, n, n_pad, blk):
    grid = n_pad // blk

    def kern(p0_ref, p1_ref, x_ref, wrel_ref, brel_ref, wroot_ref,
             h0_ref, s_ref, acc_ref):
        pid = pl.program_id(0)

        @pl.when(pid == 0)
        def _():
            acc_ref[...] = jnp.zeros_like(acc_ref)

        wrel_v = wrel_ref[0:1, 0:1]
        wroot_v = wroot_ref[0:1, 0:1]
        brel_v = brel_ref[0:1, 0:1]
        h0 = (p0_ref[...] + p1_ref[...]) * wrel_v + brel_v \
            + x_ref[...] * wroot_v
        h0_ref[...] = h0
        colid = lax.broadcasted_iota(jnp.int32, (1, blk), 1) + pid * blk
        hm = jnp.where(colid < n, h0, 0.0)
        acc_ref[0:1, 0:1] += jnp.sum(hm, axis=1, keepdims=True)
        acc_ref[1:2, 0:1] += jnp.sum(hm * hm, axis=1, keepdims=True)

        @pl.when(pid == grid - 1)
        def _():
            s_ref[...] = acc_ref[...]

    return pl.pallas_call(
        kern,
        grid=(grid,),
        in_specs=[
            pl.BlockSpec((1, blk), lambda i: (0, i)),
            pl.BlockSpec((1, blk), lambda i: (0, i)),
            pl.BlockSpec((1, blk), lambda i: (0, i)),
            pl.BlockSpec((1, 1), lambda i: (0, 0)),
            pl.BlockSpec((1, 1), lambda i: (0, 0)),
            pl.BlockSpec((1, 1), lambda i: (0, 0)),
        ],
        out_specs=[
            pl.BlockSpec((1, blk), lambda i: (0, i)),
            pl.BlockSpec((2, 1), lambda i: (0, 0)),
        ],
        out_shape=[
            jax.ShapeDtypeStruct((1, n_pad), jnp.float32),
            jax.ShapeDtypeStruct((2, 1), jnp.float32),
        ],
        scratch_shapes=[pltpu.VMEM((2, 1), jnp.float32)],
    )(p0.reshape(1, n_pad), p1.reshape(1, n_pad), xw, wrel,
      brel.reshape(1, 1), wroot)


def _layer2_pass(h0w, s0, w_in_col, g_col, be_col, w1t, b1_col,
                 n, n_pad, h, blk):
    grid = n_pad // blk
    extra = float(n_pad - n)

    def kern(h0_ref, s0_ref, win_ref, g_ref, be_ref, w_ref, b_ref,
             y_ref, s_ref, acc_ref):
        pid = pl.program_id(0)

        @pl.when(pid == 0)
        def _():
            acc_ref[...] = jnp.zeros_like(acc_ref)

        mu0 = s0_ref[0:1, 0:1] / n
        var0 = s0_ref[1:2, 0:1] / n - mu0 * mu0
        win = win_ref[...]
        c = win * g_ref[...] * lax.rsqrt(var0 * win * win + _EPS)
        u = h0_ref[...] - mu0
        a = jnp.maximum(c * u + be_ref[...], 0.0)
        colid = lax.broadcasted_iota(jnp.int32, (1, blk), 1) + pid * blk
        a = jnp.where(colid < n, a, 0.0)
        y = jnp.dot(w_ref[...], a, preferred_element_type=jnp.float32) \
            + b_ref[...]
        y_ref[...] = y
        acc_ref[:, 0:1] += jnp.sum(y, axis=1, keepdims=True)
        acc_ref[:, 1:2] += jnp.sum(y * y, axis=1, keepdims=True)

        @pl.when(pid == grid - 1)
        def _():
            b = b_ref[...]
            s_ref[...] = acc_ref[...] - jnp.concatenate(
                [b, b * b], axis=1) * extra

    return pl.pallas_call(
        kern,
        grid=(grid,),
        in_specs=[
            pl.BlockSpec((1, blk), lambda i: (0, i)),
            pl.BlockSpec((2, 1), lambda i: (0, 0)),
            pl.BlockSpec((h, 1), lambda i: (0, 0)),
            pl.BlockSpec((h, 1), lambda i: (0, 0)),
            pl.BlockSpec((h, 1), lambda i: (0, 0)),
            pl.BlockSpec((h, h), lambda i: (0, 0)),
            pl.BlockSpec((h, 1), lambda i: (0, 0)),
        ],
        out_specs=[
            pl.BlockSpec((h, blk), lambda i: (0, i)),
            pl.BlockSpec((h, 2), lambda i: (0, 0)),
        ],
        out_shape=[
            jax.ShapeDtypeStruct((h, n_pad), jnp.float32),
            jax.ShapeDtypeStruct((h, 2), jnp.float32),
        ],
        scratch_shapes=[pltpu.VMEM((h, 2), jnp.float32)],
    )(h0w, s0, w_in_col, g_col, be_col, w1t, b1_col)


def _hidden_pass(y_in, s_in, g_col, be_col, wt, b_col, n, n_pad, h, blk):
    grid = n_pad // blk
    extra = float(n_pad - n)

    def kern(y_in_ref, s_in_ref, g_ref, be_ref, w_ref, b_ref,
             y_ref, s_ref, acc_ref):
        pid = pl.program_id(0)

        @pl.when(pid == 0)
        def _():
            acc_ref[...] = jnp.zeros_like(acc_ref)

        mean = s_in_ref[:, 0:1] / n
        var = s_in_ref[:, 1:2] / n - mean * mean
        scale = lax.rsqrt(var + _EPS) * g_ref[...]
        a = jnp.maximum((y_in_ref[...] - mean) * scale + be_ref[...], 0.0)
        colid = lax.broadcasted_iota(jnp.int32, (1, blk), 1) + pid * blk
        a = jnp.where(colid < n, a, 0.0)
        y = jnp.dot(w_ref[...], a, preferred_element_type=jnp.float32) \
            + b_ref[...]
        y_ref[...] = y
        acc_ref[:, 0:1] += jnp.sum(y, axis=1, keepdims=True)
        acc_ref[:, 1:2] += jnp.sum(y * y, axis=1, keepdims=True)

        @pl.when(pid == grid - 1)
        def _():
            b2 = b_ref[...]
            s_ref[...] = acc_ref[...] - jnp.concatenate(
                [b2, b2 * b2], axis=1) * extra

    return pl.pallas_call(
        kern,
        grid=(grid,),
        in_specs=[
            pl.BlockSpec((h, blk), lambda i: (0, i)),
            pl.BlockSpec((h, 2), lambda i: (0, 0)),
            pl.BlockSpec((h, 1), lambda i: (0, 0)),
            pl.BlockSpec((h, 1), lambda i: (0, 0)),
            pl.BlockSpec((h, h), lambda i: (0, 0)),
            pl.BlockSpec((h, 1), lambda i: (0, 0)),
        ],
        out_specs=[
            pl.BlockSpec((h, blk), lambda i: (0, i)),
            pl.BlockSpec((h, 2), lambda i: (0, 0)),
        ],
        out_shape=[
            jax.ShapeDtypeStruct((h, n_pad), jnp.float32),
            jax.ShapeDtypeStruct((h, 2), jnp.float32),
        ],
        scratch_shapes=[pltpu.VMEM((h, 2), jnp.float32)],
    )(y_in, s_in, g_col, be_col, wt, b_col)


def _final_pass(y_in, s_in, g_col, be_col, w_out_row, b_out, n, n_pad, h,
                blk):
    grid = n_pad // blk

    def kern(y_in_ref, s_in_ref, g_ref, be_ref, w_ref, b_ref, o_ref):
        mean = s_in_ref[:, 0:1] / n
        var = s_in_ref[:, 1:2] / n - mean * mean
        scale = lax.rsqrt(var + _EPS) * g_ref[...]
        a = jnp.maximum((y_in_ref[...] - mean) * scale + be_ref[...], 0.0)
        z = jnp.dot(w_ref[...], a, preferred_element_type=jnp.float32) \
            + b_ref[0:1, 0:1]
        o_ref[...] = jax.nn.sigmoid(z)

    return pl.pallas_call(
        kern,
        grid=(grid,),
        in_specs=[
            pl.BlockSpec((h, blk), lambda i: (0, i)),
            pl.BlockSpec((h, 2), lambda i: (0, 0)),
            pl.BlockSpec((h, 1), lambda i: (0, 0)),
            pl.BlockSpec((h, 1), lambda i: (0, 0)),
            pl.BlockSpec((1, h), lambda i: (0, 0)),
            pl.BlockSpec((1, 1), lambda i: (0, 0)),
        ],
        out_specs=pl.BlockSpec((1, blk), lambda i: (0, i)),
        out_shape=jax.ShapeDtypeStruct((1, n_pad), jnp.float32),
    )(y_in, s_in, g_col, be_col, w_out_row, b_out)


def kernel(x, edge_index, Wrel, brel, Wroot, W_in, b_in, g_in, be_in,
           W_hid, b_hid, g_hid, be_hid, W_out, b_out):
    n = x.shape[0]
    num_edges = edge_index.shape[1]
    h = W_in.shape[1]
    blk = 2048
    n_pad = ((n + blk - 1) // blk) * blk

    xf = x[:, 0]
    x_pad = jnp.pad(xf, (0, n_pad - n))
    zeros = jnp.zeros((n_pad,), jnp.float32)
    src = jnp.reshape(edge_index[0], (num_edges,))
    dst = jnp.reshape(edge_index[1], (num_edges,))
    p0, p1 = _seg_sum_sc(x_pad, src, dst, zeros, n_pad, num_edges)

    xw = x_pad.reshape(1, n_pad)
    h0w, s0 = _h0_pass(p0, p1, xw, Wrel, brel, Wroot, n, n_pad,
                       n_pad // 8)
    y, s = _layer2_pass(h0w, s0, W_in.reshape(h, 1), g_in.reshape(h, 1),
                        be_in.reshape(h, 1), W_hid[0].T,
                        b_hid[0].reshape(h, 1), n, n_pad, h, blk)
    for i in range(1, 6):
        y, s = _hidden_pass(y, s, g_hid[i - 1].reshape(h, 1),
                            be_hid[i - 1].reshape(h, 1), W_hid[i].T,
                            b_hid[i].reshape(h, 1), n, n_pad, h, blk)
    outw = _final_pass(y, s, g_hid[5].reshape(h, 1),
                       be_hid[5].reshape(h, 1), W_out.reshape(1, h),
                       b_out.reshape(1, 1), n, n_pad, h, blk)
    return outw[0, :n].reshape(n, 1)


# speedup vs baseline: 1.2096x; 1.2096x over previous
import functools

import jax
import jax.numpy as jnp
from jax import lax
from jax.experimental import pallas as pl
from jax.experimental.pallas import tpu as pltpu
from jax.experimental.pallas import tpu_sc as plsc

_EPS = 1e-5


def _seg_sum_sc(x_pad, src, dst, zeros, n_pad, num_edges):
    num_workers = 32
    tiles = 16
    chunk = 6400
    total_chunks = num_edges // chunk
    q, r = divmod(total_chunks, num_workers)
    all_ch = q + (1 if r else 0)
    all_ch += all_ch % 2
    slice_n = n_pad // tiles

    mesh = plsc.VectorSubcoreMesh(core_axis_name="c", subcore_axis_name="s")

    @functools.partial(
        pl.kernel,
        mesh=mesh,
        out_type=[
            jax.ShapeDtypeStruct((n_pad,), jnp.float32),
            jax.ShapeDtypeStruct((n_pad,), jnp.float32),
        ],
        scratch_types=[
            pltpu.VMEM((chunk,), jnp.int32),
            pltpu.VMEM((chunk,), jnp.int32),
            pltpu.VMEM((chunk,), jnp.int32),
            pltpu.VMEM((chunk,), jnp.int32),
            pltpu.VMEM((chunk,), jnp.float32),
            pltpu.VMEM((chunk,), jnp.float32),
            pltpu.VMEM_SHARED((n_pad,), jnp.float32),
            pltpu.VMEM_SHARED((n_pad,), jnp.float32),
            pltpu.SemaphoreType.DMA,
            pltpu.SemaphoreType.DMA,
            pltpu.SemaphoreType.DMA,
            pltpu.SemaphoreType.DMA,
            pltpu.SemaphoreType.DMA,
            pltpu.SemaphoreType.DMA,
        ],
    )
    def seg_kernel(x_hbm, src_hbm, dst_hbm, z_hbm, p0_hbm, p1_hbm,
                   srcbuf0, srcbuf1, dstbuf0, dstbuf1, valbuf0, valbuf1,
                   x_sh, acc_sh, sem_i0, sem_i1, sem_g0, sem_g1, sem_s0,
                   sem_s1):
        sc = lax.axis_index("c")
        sid = lax.axis_index("s")
        base_n = sid * slice_n
        pltpu.sync_copy(x_hbm.at[pl.ds(base_n, slice_n)],
                        x_sh.at[pl.ds(base_n, slice_n)])
        pltpu.sync_copy(z_hbm.at[pl.ds(base_n, slice_n)],
                        acc_sh.at[pl.ds(base_n, slice_n)])
        plsc.subcore_barrier()

        wid = sc * tiles + sid
        nch = q + jnp.where(wid < r, 1, 0)

        def chunk_off(j):
            joff = jnp.where(j < nch, j, 0)
            return (wid + joff * num_workers) * chunk

        pltpu.sync_copy(z_hbm.at[pl.ds(0, chunk)], valbuf0)
        pltpu.sync_copy(z_hbm.at[pl.ds(0, chunk)], valbuf1)
        pltpu.sync_copy(dst_hbm.at[pl.ds(wid * chunk, chunk)], dstbuf0)
        pltpu.sync_copy(dst_hbm.at[pl.ds(wid * chunk, chunk)], dstbuf1)
        pltpu.async_copy(valbuf0, acc_sh.at[dstbuf0], sem_s0, add=True)
        pltpu.async_copy(valbuf1, acc_sh.at[dstbuf1], sem_s1, add=True)

        def body(k, _):
            j0 = 2 * k
            j1 = 2 * k + 1
            pltpu.make_async_copy(valbuf0, acc_sh.at[dstbuf0], sem_s0).wait()
            off0 = chunk_off(j0)
            ds0 = pltpu.async_copy(src_hbm.at[pl.ds(off0, chunk)], srcbuf0,
                                   sem_i0)
            dd0 = pltpu.async_copy(dst_hbm.at[pl.ds(off0, chunk)], dstbuf0,
                                   sem_i0)
            pltpu.make_async_copy(valbuf1, acc_sh.at[dstbuf1], sem_s1).wait()
            off1 = chunk_off(j1)
            ds1 = pltpu.async_copy(src_hbm.at[pl.ds(off1, chunk)], srcbuf1,
                                   sem_i1)
            dd1 = pltpu.async_copy(dst_hbm.at[pl.ds(off1, chunk)], dstbuf1,
                                   sem_i1)
            ds0.wait()
            dd0.wait()
            g0 = pltpu.async_copy(x_sh.at[srcbuf0], valbuf0, sem_g0)
            ds1.wait()
            dd1.wait()
            g1 = pltpu.async_copy(x_sh.at[srcbuf1], valbuf1, sem_g1)
            g0.wait()

            @pl.when(j0 >= nch)
            def _():
                pltpu.sync_copy(z_hbm.at[pl.ds(0, chunk)], valbuf0)

            pltpu.async_copy(valbuf0, acc_sh.at[dstbuf0], sem_s0, add=True)
            g1.wait()

            @pl.when(j1 >= nch)
            def _():
                pltpu.sync_copy(z_hbm.at[pl.ds(0, chunk)], valbuf1)

            pltpu.async_copy(valbuf1, acc_sh.at[dstbuf1], sem_s1, add=True)
            return _

        lax.fori_loop(0, all_ch // 2, body, None)
        pltpu.make_async_copy(valbuf0, acc_sh.at[dstbuf0], sem_s0).wait()
        pltpu.make_async_copy(valbuf1, acc_sh.at[dstbuf1], sem_s1).wait()
        plsc.subcore_barrier()

        @pl.when(sc == 0)
        def _():
            pltpu.sync_copy(acc_sh.at[pl.ds(base_n, slice_n)],
                            p0_hbm.at[pl.ds(base_n, slice_n)])

        @pl.when(sc == 1)
        def _():
            pltpu.sync_copy(acc_sh.at[pl.ds(base_n, slice_n)],
                            p1_hbm.at[pl.ds(base_n, slice_n)])

    return seg_kernel(x_pad, src, dst, zeros)


def _h0_pass(p0, p1, xw, wrel, brel, wroot, n, n_pad, blk):
    grid = n_pad // blk

    def kern(p0_ref, p1_ref, x_ref, wrel_ref, brel_ref, wroot_ref,
             h0_ref, s_ref, acc_ref):
        pid = pl.program_id(0)

        @pl.when(pid == 0)
        def _():
            acc_ref[...] = jnp.zeros_like(acc_ref)

        wrel_v = wrel_ref[0:1, 0:1]
        wroot_v = wroot_ref[0:1, 0:1]
        brel_v = brel_ref[0:1, 0:1]
        h0 = (p0_ref[...] + p1_ref[...]) * wrel_v + brel_v \
            + x_ref[...] * wroot_v
        h0_ref[...] = h0
        colid = lax.broadcasted_iota(jnp.int32, (1, blk), 1) + pid * blk
        hm = jnp.where(colid < n, h0, 0.0)
        acc_ref[0:1, 0:1] += jnp.sum(hm, axis=1, keepdims=True)
        acc_ref[1:2, 0:1] += jnp.sum(hm * hm, axis=1, keepdims=True)

        @pl.when(pid == grid - 1)
        def _():
            s_ref[...] = acc_ref[...]

    return pl.pallas_call(
        kern,
        grid=(grid,),
        in_specs=[
            pl.BlockSpec((1, blk), lambda i: (0, i)),
            pl.BlockSpec((1, blk), lambda i: (0, i)),
            pl.BlockSpec((1, blk), lambda i: (0, i)),
            pl.BlockSpec((1, 1), lambda i: (0, 0)),
            pl.BlockSpec((1, 1), lambda i: (0, 0)),
            pl.BlockSpec((1, 1), lambda i: (0, 0)),
        ],
        out_specs=[
            pl.BlockSpec((1, blk), lambda i: (0, i)),
            pl.BlockSpec((2, 1), lambda i: (0, 0)),
        ],
        out_shape=[
            jax.ShapeDtypeStruct((1, n_pad), jnp.float32),
            jax.ShapeDtypeStruct((2, 1), jnp.float32),
        ],
        scratch_shapes=[pltpu.VMEM((2, 1), jnp.float32)],
    )(p0.reshape(1, n_pad), p1.reshape(1, n_pad), xw, wrel,
      brel.reshape(1, 1), wroot)


def _layer2_pass(h0w, s0, w_in_col, g_col, be_col, w1t, b1_col,
                 n, n_pad, h, blk):
    grid = n_pad // blk
    extra = float(n_pad - n)

    def kern(h0_ref, s0_ref, win_ref, g_ref, be_ref, w_ref, b_ref,
             y_ref, s_ref, acc_ref):
        pid = pl.program_id(0)

        @pl.when(pid == 0)
        def _():
            acc_ref[...] = jnp.zeros_like(acc_ref)

        mu0 = s0_ref[0:1, 0:1] / n
        var0 = s0_ref[1:2, 0:1] / n - mu0 * mu0
        win = win_ref[...]
        c = win * g_ref[...] * lax.rsqrt(var0 * win * win + _EPS)
        u = h0_ref[...] - mu0
        a = jnp.maximum(c * u + be_ref[...], 0.0)
        colid = lax.broadcasted_iota(jnp.int32, (1, blk), 1) + pid * blk
        a = jnp.where(colid < n, a, 0.0)
        y = jnp.dot(w_ref[...], a, preferred_element_type=jnp.float32) \
            + b_ref[...]
        y_ref[...] = y
        acc_ref[:, 0:1] += jnp.sum(y, axis=1, keepdims=True)
        acc_ref[:, 1:2] += jnp.sum(y * y, axis=1, keepdims=True)

        @pl.when(pid == grid - 1)
        def _():
            b = b_ref[...]
            s_ref[...] = acc_ref[...] - jnp.concatenate(
                [b, b * b], axis=1) * extra

    return pl.pallas_call(
        kern,
        grid=(grid,),
        in_specs=[
            pl.BlockSpec((1, blk), lambda i: (0, i)),
            pl.BlockSpec((2, 1), lambda i: (0, 0)),
            pl.BlockSpec((h, 1), lambda i: (0, 0)),
            pl.BlockSpec((h, 1), lambda i: (0, 0)),
            pl.BlockSpec((h, 1), lambda i: (0, 0)),
            pl.BlockSpec((h, h), lambda i: (0, 0)),
            pl.BlockSpec((h, 1), lambda i: (0, 0)),
        ],
        out_specs=[
            pl.BlockSpec((h, blk), lambda i: (0, i)),
            pl.BlockSpec((h, 2), lambda i: (0, 0)),
        ],
        out_shape=[
            jax.ShapeDtypeStruct((h, n_pad), jnp.float32),
            jax.ShapeDtypeStruct((h, 2), jnp.float32),
        ],
        scratch_shapes=[pltpu.VMEM((h, 2), jnp.float32)],
    )(h0w, s0, w_in_col, g_col, be_col, w1t, b1_col)


def _hidden_pass(y_in, s_in, g_col, be_col, wt, b_col, n, n_pad, h, blk):
    grid = n_pad // blk
    extra = float(n_pad - n)

    def kern(y_in_ref, s_in_ref, g_ref, be_ref, w_ref, b_ref,
             y_ref, s_ref, acc_ref):
        pid = pl.program_id(0)

        @pl.when(pid == 0)
        def _():
            acc_ref[...] = jnp.zeros_like(acc_ref)

        mean = s_in_ref[:, 0:1] / n
        var = s_in_ref[:, 1:2] / n - mean * mean
        scale = lax.rsqrt(var + _EPS) * g_ref[...]
        a = jnp.maximum((y_in_ref[...] - mean) * scale + be_ref[...], 0.0)
        colid = lax.broadcasted_iota(jnp.int32, (1, blk), 1) + pid * blk
        a = jnp.where(colid < n, a, 0.0)
        y = jnp.dot(w_ref[...], a, preferred_element_type=jnp.float32) \
            + b_ref[...]
        y_ref[...] = y
        acc_ref[:, 0:1] += jnp.sum(y, axis=1, keepdims=True)
        acc_ref[:, 1:2] += jnp.sum(y * y, axis=1, keepdims=True)

        @pl.when(pid == grid - 1)
        def _():
            b2 = b_ref[...]
            s_ref[...] = acc_ref[...] - jnp.concatenate(
                [b2, b2 * b2], axis=1) * extra

    return pl.pallas_call(
        kern,
        grid=(grid,),
        in_specs=[
            pl.BlockSpec((h, blk), lambda i: (0, i)),
            pl.BlockSpec((h, 2), lambda i: (0, 0)),
            pl.BlockSpec((h, 1), lambda i: (0, 0)),
            pl.BlockSpec((h, 1), lambda i: (0, 0)),
            pl.BlockSpec((h, h), lambda i: (0, 0)),
            pl.BlockSpec((h, 1), lambda i: (0, 0)),
        ],
        out_specs=[
            pl.BlockSpec((h, blk), lambda i: (0, i)),
            pl.BlockSpec((h, 2), lambda i: (0, 0)),
        ],
        out_shape=[
            jax.ShapeDtypeStruct((h, n_pad), jnp.float32),
            jax.ShapeDtypeStruct((h, 2), jnp.float32),
        ],
        scratch_shapes=[pltpu.VMEM((h, 2), jnp.float32)],
    )(y_in, s_in, g_col, be_col, wt, b_col)


def _final_pass(y_in, s_in, g_col, be_col, w_out_row, b_out, n, n_pad, h,
                blk):
    grid = n_pad // blk

    def kern(y_in_ref, s_in_ref, g_ref, be_ref, w_ref, b_ref, o_ref):
        mean = s_in_ref[:, 0:1] / n
        var = s_in_ref[:, 1:2] / n - mean * mean
        scale = lax.rsqrt(var + _EPS) * g_ref[...]
        a = jnp.maximum((y_in_ref[...] - mean) * scale + be_ref[...], 0.0)
        z = jnp.dot(w_ref[...], a, preferred_element_type=jnp.float32) \
            + b_ref[0:1, 0:1]
        o_ref[...] = jax.nn.sigmoid(z)

    return pl.pallas_call(
        kern,
        grid=(grid,),
        in_specs=[
            pl.BlockSpec((h, blk), lambda i: (0, i)),
            pl.BlockSpec((h, 2), lambda i: (0, 0)),
            pl.BlockSpec((h, 1), lambda i: (0, 0)),
            pl.BlockSpec((h, 1), lambda i: (0, 0)),
            pl.BlockSpec((1, h), lambda i: (0, 0)),
            pl.BlockSpec((1, 1), lambda i: (0, 0)),
        ],
        out_specs=pl.BlockSpec((1, blk), lambda i: (0, i)),
        out_shape=jax.ShapeDtypeStruct((1, n_pad), jnp.float32),
    )(y_in, s_in, g_col, be_col, w_out_row, b_out)


def kernel(x, edge_index, Wrel, brel, Wroot, W_in, b_in, g_in, be_in,
           W_hid, b_hid, g_hid, be_hid, W_out, b_out):
    n = x.shape[0]
    num_edges = edge_index.shape[1]
    h = W_in.shape[1]
    blk = 2048
    n_pad = ((n + blk - 1) // blk) * blk

    xf = x[:, 0]
    x_pad = jnp.pad(xf, (0, n_pad - n))
    zeros = jnp.zeros((n_pad,), jnp.float32)
    src = jnp.reshape(edge_index[0], (num_edges,))
    dst = jnp.reshape(edge_index[1], (num_edges,))
    p0, p1 = _seg_sum_sc(x_pad, src, dst, zeros, n_pad, num_edges)

    xw = x_pad.reshape(1, n_pad)
    h0w, s0 = _h0_pass(p0, p1, xw, Wrel, brel, Wroot, n, n_pad,
                       n_pad // 8)
    y, s = _layer2_pass(h0w, s0, W_in.reshape(h, 1), g_in.reshape(h, 1),
                        be_in.reshape(h, 1), W_hid[0].T,
                        b_hid[0].reshape(h, 1), n, n_pad, h, blk)
    for i in range(1, 6):
        y, s = _hidden_pass(y, s, g_hid[i - 1].reshape(h, 1),
                            be_hid[i - 1].reshape(h, 1), W_hid[i].T,
                            b_hid[i].reshape(h, 1), n, n_pad, h, blk)
    outw = _final_pass(y, s, g_hid[5].reshape(h, 1),
                       be_hid[5].reshape(h, 1), W_out.reshape(1, h),
                       b_out.reshape(1, 1), n, n_pad, h, blk)
    return outw[0, :n].reshape(n, 1)


# speedup vs baseline: 1.3894x; 1.1486x over previous
import functools

import jax
import jax.numpy as jnp
from jax import lax
from jax.experimental import pallas as pl
from jax.experimental.pallas import tpu as pltpu
from jax.experimental.pallas import tpu_sc as plsc

_EPS = 1e-5


def _seg_sum_sc(x_pad, src, dst, zeros, n_pad, num_edges):
    num_workers = 32
    tiles = 16
    chunk = 12800
    total_chunks = num_edges // chunk
    q, r = divmod(total_chunks, num_workers)
    all_ch = q + (1 if r else 0)
    all_ch += all_ch % 2
    slice_n = n_pad // tiles

    mesh = plsc.VectorSubcoreMesh(core_axis_name="c", subcore_axis_name="s")

    @functools.partial(
        pl.kernel,
        mesh=mesh,
        out_type=[
            jax.ShapeDtypeStruct((n_pad,), jnp.float32),
            jax.ShapeDtypeStruct((n_pad,), jnp.float32),
        ],
        scratch_types=[
            pltpu.VMEM((chunk,), jnp.int32),
            pltpu.VMEM((chunk,), jnp.int32),
            pltpu.VMEM((chunk,), jnp.int32),
            pltpu.VMEM((chunk,), jnp.int32),
            pltpu.VMEM((chunk,), jnp.float32),
            pltpu.VMEM((chunk,), jnp.float32),
            pltpu.VMEM_SHARED((n_pad,), jnp.float32),
            pltpu.VMEM_SHARED((n_pad,), jnp.float32),
            pltpu.SemaphoreType.DMA,
            pltpu.SemaphoreType.DMA,
            pltpu.SemaphoreType.DMA,
            pltpu.SemaphoreType.DMA,
            pltpu.SemaphoreType.DMA,
            pltpu.SemaphoreType.DMA,
        ],
    )
    def seg_kernel(x_hbm, src_hbm, dst_hbm, z_hbm, p0_hbm, p1_hbm,
                   srcbuf0, srcbuf1, dstbuf0, dstbuf1, valbuf0, valbuf1,
                   x_sh, acc_sh, sem_i0, sem_i1, sem_g0, sem_g1, sem_s0,
                   sem_s1):
        sc = lax.axis_index("c")
        sid = lax.axis_index("s")
        base_n = sid * slice_n
        pltpu.sync_copy(x_hbm.at[pl.ds(base_n, slice_n)],
                        x_sh.at[pl.ds(base_n, slice_n)])
        pltpu.sync_copy(z_hbm.at[pl.ds(base_n, slice_n)],
                        acc_sh.at[pl.ds(base_n, slice_n)])
        plsc.subcore_barrier()

        wid = sc * tiles + sid
        nch = q + jnp.where(wid < r, 1, 0)

        def chunk_off(j):
            joff = jnp.where(j < nch, j, 0)
            return (wid + joff * num_workers) * chunk

        pltpu.sync_copy(z_hbm.at[pl.ds(0, chunk)], valbuf0)
        pltpu.sync_copy(z_hbm.at[pl.ds(0, chunk)], valbuf1)
        pltpu.sync_copy(dst_hbm.at[pl.ds(wid * chunk, chunk)], dstbuf0)
        pltpu.sync_copy(dst_hbm.at[pl.ds(wid * chunk, chunk)], dstbuf1)
        pltpu.async_copy(valbuf0, acc_sh.at[dstbuf0], sem_s0, add=True)
        pltpu.async_copy(valbuf1, acc_sh.at[dstbuf1], sem_s1, add=True)

        def body(k, _):
            j0 = 2 * k
            j1 = 2 * k + 1
            pltpu.make_async_copy(valbuf0, acc_sh.at[dstbuf0], sem_s0).wait()
            off0 = chunk_off(j0)
            ds0 = pltpu.async_copy(src_hbm.at[pl.ds(off0, chunk)], srcbuf0,
                                   sem_i0)
            dd0 = pltpu.async_copy(dst_hbm.at[pl.ds(off0, chunk)], dstbuf0,
                                   sem_i0)
            pltpu.make_async_copy(valbuf1, acc_sh.at[dstbuf1], sem_s1).wait()
            off1 = chunk_off(j1)
            ds1 = pltpu.async_copy(src_hbm.at[pl.ds(off1, chunk)], srcbuf1,
                                   sem_i1)
            dd1 = pltpu.async_copy(dst_hbm.at[pl.ds(off1, chunk)], dstbuf1,
                                   sem_i1)
            ds0.wait()
            dd0.wait()
            g0 = pltpu.async_copy(x_sh.at[srcbuf0], valbuf0, sem_g0)
            ds1.wait()
            dd1.wait()
            g1 = pltpu.async_copy(x_sh.at[srcbuf1], valbuf1, sem_g1)
            g0.wait()

            @pl.when(j0 >= nch)
            def _():
                pltpu.sync_copy(z_hbm.at[pl.ds(0, chunk)], valbuf0)

            pltpu.async_copy(valbuf0, acc_sh.at[dstbuf0], sem_s0, add=True)
            g1.wait()

            @pl.when(j1 >= nch)
            def _():
                pltpu.sync_copy(z_hbm.at[pl.ds(0, chunk)], valbuf1)

            pltpu.async_copy(valbuf1, acc_sh.at[dstbuf1], sem_s1, add=True)
            return _

        lax.fori_loop(0, all_ch // 2, body, None)
        pltpu.make_async_copy(valbuf0, acc_sh.at[dstbuf0], sem_s0).wait()
        pltpu.make_async_copy(valbuf1, acc_sh.at[dstbuf1], sem_s1).wait()
        plsc.subcore_barrier()

        @pl.when(sc == 0)
        def _():
            pltpu.sync_copy(acc_sh.at[pl.ds(base_n, slice_n)],
                            p0_hbm.at[pl.ds(base_n, slice_n)])

        @pl.when(sc == 1)
        def _():
            pltpu.sync_copy(acc_sh.at[pl.ds(base_n, slice_n)],
                            p1_hbm.at[pl.ds(base_n, slice_n)])

    return seg_kernel(x_pad, src, dst, zeros)


def _h0_pass(p0, p1, xw, wrel, brel, wroot, n, n_pad, blk):
    grid = n_pad // blk

    def kern(p0_ref, p1_ref, x_ref, wrel_ref, brel_ref, wroot_ref,
             h0_ref, s_ref, acc_ref):
        pid = pl.program_id(0)

        @pl.when(pid == 0)
        def _():
            acc_ref[...] = jnp.zeros_like(acc_ref)

        wrel_v = wrel_ref[0:1, 0:1]
        wroot_v = wroot_ref[0:1, 0:1]
        brel_v = brel_ref[0:1, 0:1]
        h0 = (p0_ref[...] + p1_ref[...]) * wrel_v + brel_v \
            + x_ref[...] * wroot_v
        h0_ref[...] = h0
        colid = lax.broadcasted_iota(jnp.int32, (1, blk), 1) + pid * blk
        hm = jnp.where(colid < n, h0, 0.0)
        acc_ref[0:1, 0:1] += jnp.sum(hm, axis=1, keepdims=True)
        acc_ref[1:2, 0:1] += jnp.sum(hm * hm, axis=1, keepdims=True)

        @pl.when(pid == grid - 1)
        def _():
            s_ref[...] = acc_ref[...]

    return pl.pallas_call(
        kern,
        grid=(grid,),
        in_specs=[
            pl.BlockSpec((1, blk), lambda i: (0, i)),
            pl.BlockSpec((1, blk), lambda i: (0, i)),
            pl.BlockSpec((1, blk), lambda i: (0, i)),
            pl.BlockSpec((1, 1), lambda i: (0, 0)),
            pl.BlockSpec((1, 1), lambda i: (0, 0)),
            pl.BlockSpec((1, 1), lambda i: (0, 0)),
        ],
        out_specs=[
            pl.BlockSpec((1, blk), lambda i: (0, i)),
            pl.BlockSpec((2, 1), lambda i: (0, 0)),
        ],
        out_shape=[
            jax.ShapeDtypeStruct((1, n_pad), jnp.float32),
            jax.ShapeDtypeStruct((2, 1), jnp.float32),
        ],
        scratch_shapes=[pltpu.VMEM((2, 1), jnp.float32)],
    )(p0.reshape(1, n_pad), p1.reshape(1, n_pad), xw, wrel,
      brel.reshape(1, 1), wroot)


def _layer2_pass(h0w, s0, w_in_col, g_col, be_col, w1t, b1_col,
                 n, n_pad, h, blk):
    grid = n_pad // blk
    extra = float(n_pad - n)

    def kern(h0_ref, s0_ref, win_ref, g_ref, be_ref, w_ref, b_ref,
             y_ref, s_ref, acc_ref):
        pid = pl.program_id(0)

        @pl.when(pid == 0)
        def _():
            acc_ref[...] = jnp.zeros_like(acc_ref)

        mu0 = s0_ref[0:1, 0:1] / n
        var0 = s0_ref[1:2, 0:1] / n - mu0 * mu0
        win = win_ref[...]
        c = win * g_ref[...] * lax.rsqrt(var0 * win * win + _EPS)
        u = h0_ref[...] - mu0
        a = jnp.maximum(c * u + be_ref[...], 0.0)
        colid = lax.broadcasted_iota(jnp.int32, (1, blk), 1) + pid * blk
        a = jnp.where(colid < n, a, 0.0)
        y = jnp.dot(w_ref[...], a, preferred_element_type=jnp.float32) \
            + b_ref[...]
        y_ref[...] = y
        acc_ref[:, 0:1] += jnp.sum(y, axis=1, keepdims=True)
        acc_ref[:, 1:2] += jnp.sum(y * y, axis=1, keepdims=True)

        @pl.when(pid == grid - 1)
        def _():
            b = b_ref[...]
            s_ref[...] = acc_ref[...] - jnp.concatenate(
                [b, b * b], axis=1) * extra

    return pl.pallas_call(
        kern,
        grid=(grid,),
        in_specs=[
            pl.BlockSpec((1, blk), lambda i: (0, i)),
            pl.BlockSpec((2, 1), lambda i: (0, 0)),
            pl.BlockSpec((h, 1), lambda i: (0, 0)),
            pl.BlockSpec((h, 1), lambda i: (0, 0)),
            pl.BlockSpec((h, 1), lambda i: (0, 0)),
            pl.BlockSpec((h, h), lambda i: (0, 0)),
            pl.BlockSpec((h, 1), lambda i: (0, 0)),
        ],
        out_specs=[
            pl.BlockSpec((h, blk), lambda i: (0, i)),
            pl.BlockSpec((h, 2), lambda i: (0, 0)),
        ],
        out_shape=[
            jax.ShapeDtypeStruct((h, n_pad), jnp.float32),
            jax.ShapeDtypeStruct((h, 2), jnp.float32),
        ],
        scratch_shapes=[pltpu.VMEM((h, 2), jnp.float32)],
    )(h0w, s0, w_in_col, g_col, be_col, w1t, b1_col)


def _hidden_pass(y_in, s_in, g_col, be_col, wt, b_col, n, n_pad, h, blk):
    grid = n_pad // blk
    extra = float(n_pad - n)

    def kern(y_in_ref, s_in_ref, g_ref, be_ref, w_ref, b_ref,
             y_ref, s_ref, acc_ref):
        pid = pl.program_id(0)

        @pl.when(pid == 0)
        def _():
            acc_ref[...] = jnp.zeros_like(acc_ref)

        mean = s_in_ref[:, 0:1] / n
        var = s_in_ref[:, 1:2] / n - mean * mean
        scale = lax.rsqrt(var + _EPS) * g_ref[...]
        a = jnp.maximum((y_in_ref[...] - mean) * scale + be_ref[...], 0.0)
        colid = lax.broadcasted_iota(jnp.int32, (1, blk), 1) + pid * blk
        a = jnp.where(colid < n, a, 0.0)
        y = jnp.dot(w_ref[...], a, preferred_element_type=jnp.float32) \
            + b_ref[...]
        y_ref[...] = y
        acc_ref[:, 0:1] += jnp.sum(y, axis=1, keepdims=True)
        acc_ref[:, 1:2] += jnp.sum(y * y, axis=1, keepdims=True)

        @pl.when(pid == grid - 1)
        def _():
            b2 = b_ref[...]
            s_ref[...] = acc_ref[...] - jnp.concatenate(
                [b2, b2 * b2], axis=1) * extra

    return pl.pallas_call(
        kern,
        grid=(grid,),
        in_specs=[
            pl.BlockSpec((h, blk), lambda i: (0, i)),
            pl.BlockSpec((h, 2), lambda i: (0, 0)),
            pl.BlockSpec((h, 1), lambda i: (0, 0)),
            pl.BlockSpec((h, 1), lambda i: (0, 0)),
            pl.BlockSpec((h, h), lambda i: (0, 0)),
            pl.BlockSpec((h, 1), lambda i: (0, 0)),
        ],
        out_specs=[
            pl.BlockSpec((h, blk), lambda i: (0, i)),
            pl.BlockSpec((h, 2), lambda i: (0, 0)),
        ],
        out_shape=[
            jax.ShapeDtypeStruct((h, n_pad), jnp.float32),
            jax.ShapeDtypeStruct((h, 2), jnp.float32),
        ],
        scratch_shapes=[pltpu.VMEM((h, 2), jnp.float32)],
    )(y_in, s_in, g_col, be_col, wt, b_col)


def _final_pass(y_in, s_in, g_col, be_col, w_out_row, b_out, n, n_pad, h,
                blk):
    grid = n_pad // blk

    def kern(y_in_ref, s_in_ref, g_ref, be_ref, w_ref, b_ref, o_ref):
        mean = s_in_ref[:, 0:1] / n
        var = s_in_ref[:, 1:2] / n - mean * mean
        scale = lax.rsqrt(var + _EPS) * g_ref[...]
        a = jnp.maximum((y_in_ref[...] - mean) * scale + be_ref[...], 0.0)
        z = jnp.dot(w_ref[...], a, preferred_element_type=jnp.float32) \
            + b_ref[0:1, 0:1]
        o_ref[...] = jax.nn.sigmoid(z)

    return pl.pallas_call(
        kern,
        grid=(grid,),
        in_specs=[
            pl.BlockSpec((h, blk), lambda i: (0, i)),
            pl.BlockSpec((h, 2), lambda i: (0, 0)),
            pl.BlockSpec((h, 1), lambda i: (0, 0)),
            pl.BlockSpec((h, 1), lambda i: (0, 0)),
            pl.BlockSpec((1, h), lambda i: (0, 0)),
            pl.BlockSpec((1, 1), lambda i: (0, 0)),
        ],
        out_specs=pl.BlockSpec((1, blk), lambda i: (0, i)),
        out_shape=jax.ShapeDtypeStruct((1, n_pad), jnp.float32),
    )(y_in, s_in, g_col, be_col, w_out_row, b_out)


def kernel(x, edge_index, Wrel, brel, Wroot, W_in, b_in, g_in, be_in,
           W_hid, b_hid, g_hid, be_hid, W_out, b_out):
    n = x.shape[0]
    num_edges = edge_index.shape[1]
    h = W_in.shape[1]
    blk = 4096
    n_pad = ((n + blk - 1) // blk) * blk

    xf = x[:, 0]
    x_pad = jnp.pad(xf, (0, n_pad - n))
    zeros = jnp.zeros((n_pad,), jnp.float32)
    src = jnp.reshape(edge_index[0], (num_edges,))
    dst = jnp.reshape(edge_index[1], (num_edges,))
    p0, p1 = _seg_sum_sc(x_pad, src, dst, zeros, n_pad, num_edges)

    xw = x_pad.reshape(1, n_pad)
    h0w, s0 = _h0_pass(p0, p1, xw, Wrel, brel, Wroot, n, n_pad,
                       n_pad // 8)
    y, s = _layer2_pass(h0w, s0, W_in.reshape(h, 1), g_in.reshape(h, 1),
                        be_in.reshape(h, 1), W_hid[0].T,
                        b_hid[0].reshape(h, 1), n, n_pad, h, blk)
    for i in range(1, 6):
        y, s = _hidden_pass(y, s, g_hid[i - 1].reshape(h, 1),
                            be_hid[i - 1].reshape(h, 1), W_hid[i].T,
                            b_hid[i].reshape(h, 1), n, n_pad, h, blk)
    outw = _final_pass(y, s, g_hid[5].reshape(h, 1),
                       be_hid[5].reshape(h, 1), W_out.reshape(1, h),
                       b_out.reshape(1, 1), n, n_pad, h, blk)
    return outw[0, :n].reshape(n, 1)
